# Initial kernel scaffold; baseline (speedup 1.0000x reference)
import jax
import jax.numpy as jnp
from jax.experimental import pallas as pl


def kernel(inputs, embedding_weights):
    raise NotImplementedError("write your pallas kernel here")


# speedup vs baseline: 4.9717x; 4.9717x over previous
import functools

import jax
import jax.numpy as jnp
from jax import lax
from jax.experimental import pallas as pl
from jax.experimental.pallas import tpu as pltpu
from jax.experimental.pallas import tpu_sc as plsc

VOCAB = 100000
D = 128
BATCH = 4096
SEQ = 200
B = BATCH * SEQ
CHUNK = 128
G = B // CHUNK

_info = plsc.get_sparse_core_info()
NC, NS, L = _info.num_cores, _info.num_subcores, _info.num_lanes
NW = NC * NS
GPW = G // NW


def _body(table_hbm, idx_hbm, out_hbm, idx_v, rows_v, gsem):
    wid = lax.axis_index("s") * NC + lax.axis_index("c")

    def chunk_body(g, _):
        cid = wid * GPW + g
        pltpu.sync_copy(idx_hbm.at[cid], idx_v)
        pltpu.async_copy(table_hbm.at[idx_v], rows_v, gsem).wait()

        def grp_body(i, _):
            iv = idx_v[pl.ds(i * L, L)]
            nzero = plsc.all_reduce_population_count(iv == 0)[0]

            @pl.when(nzero > 0)
            def _fix():
                z = jnp.zeros((L,), jnp.float32)
                for r in range(L):

                    @pl.when(iv[r] == 0)
                    def _zero_row(row=i * L + r):
                        for c in range(D // L):
                            rows_v[row, pl.ds(c * L, L)] = z

            return 0

        lax.fori_loop(0, CHUNK // L, grp_body, 0)

        pltpu.sync_copy(rows_v, out_hbm.at[pl.ds(cid * CHUNK, CHUNK)])
        return 0

    lax.fori_loop(0, GPW, chunk_body, 0)


@functools.partial(
    pl.kernel,
    out_type=jax.ShapeDtypeStruct((B, D), jnp.float32),
    mesh=plsc.VectorSubcoreMesh(core_axis_name="c", subcore_axis_name="s"),
    compiler_params=pltpu.CompilerParams(needs_layout_passes=False),
    scratch_types=[
        pltpu.VMEM((CHUNK,), jnp.int32),
        pltpu.VMEM((CHUNK, D), jnp.float32),
        pltpu.SemaphoreType.DMA,
    ],
)
def _emb_lookup(table_hbm, idx_hbm, out_hbm, idx_v, rows_v, gsem):
    _body(table_hbm, idx_hbm, out_hbm, idx_v, rows_v, gsem)


def kernel(inputs, embedding_weights):
    idx = inputs.astype(jnp.int32).reshape(G, CHUNK)
    out = _emb_lookup(embedding_weights, idx)
    return out.reshape(BATCH, SEQ, D)


# speedup vs baseline: 9.2065x; 1.8518x over previous
import functools

import jax
import jax.numpy as jnp
from jax import lax
from jax.experimental import pallas as pl
from jax.experimental.pallas import tpu as pltpu
from jax.experimental.pallas import tpu_sc as plsc

VOCAB = 100000
D = 128
BATCH = 4096
SEQ = 200
B = BATCH * SEQ
CHUNK = 128
G = B // CHUNK

_info = plsc.get_sparse_core_info()
NC, NS, L = _info.num_cores, _info.num_subcores, _info.num_lanes
NW = NC * NS
GPW = G // NW
RPW = GPW * CHUNK

NBUF = 4
LOOK = NBUF - 1


def _body(table_hbm, idx_hbm, out_hbm, idx_all, rows_v, *sems):
    gsems, osems = sems[:NBUF], sems[NBUF:]
    wid = lax.axis_index("s") * NC + lax.axis_index("c")
    rbase = wid * RPW

    pltpu.sync_copy(idx_hbm.at[pl.ds(rbase, RPW)], idx_all)

    def fire_gather(g, b):
        isl = idx_all.at[pl.ds(g * CHUNK, CHUNK)]
        pltpu.async_copy(table_hbm.at[isl], rows_v.at[b], gsems[b])

    def wait_gather(b):
        pltpu.make_async_copy(
            table_hbm.at[pl.ds(0, CHUNK)], rows_v.at[b], gsems[b]).wait()

    def fire_out(g, b):
        osl = out_hbm.at[pl.ds(rbase + g * CHUNK, CHUNK)]
        pltpu.async_copy(rows_v.at[b], osl, osems[b])

    def wait_out(b):
        pltpu.make_async_copy(
            rows_v.at[b], out_hbm.at[pl.ds(0, CHUNK)], osems[b]).wait()

    def fix(g, b):
        def grp_body(i, _):
            iv = idx_all[pl.ds(g * CHUNK + i * L, L)]
            nzero = plsc.all_reduce_population_count(iv == 0)[0]

            @pl.when(nzero > 0)
            def _fix():
                z = jnp.zeros((L,), jnp.float32)
                for r in range(L):

                    @pl.when(iv[r] == 0)
                    def _zero_row(row=i * L + r):
                        for c in range(D // L):
                            rows_v[b, row, pl.ds(c * L, L)] = z

            return 0

        lax.fori_loop(0, CHUNK // L, grp_body, 0)

    for b in range(LOOK):
        fire_gather(jnp.int32(b), b)

    def lap(k, _):
        for b in range(NBUF):
            g = k * NBUF + b
            wait_gather(b)
            fix(g, b)
            fire_out(g, b)
            gn = g + LOOK
            b2 = (b + LOOK) % NBUF

            @pl.when(gn < GPW)
            def _issue(gn=gn, b2=b2):
                @pl.when(gn >= NBUF)
                def _drain():
                    wait_out(b2)

                fire_gather(gn, b2)

        return 0

    lax.fori_loop(0, GPW // NBUF, lap, 0)

    for b in range(NBUF):
        wait_out(b)


@functools.partial(
    pl.kernel,
    out_type=jax.ShapeDtypeStruct((B, D), jnp.float32),
    mesh=plsc.VectorSubcoreMesh(core_axis_name="c", subcore_axis_name="s"),
    compiler_params=pltpu.CompilerParams(needs_layout_passes=False),
    scratch_types=(
        [pltpu.VMEM((RPW,), jnp.int32),
         pltpu.VMEM((NBUF, CHUNK, D), jnp.float32)]
        + [pltpu.SemaphoreType.DMA] * (2 * NBUF)
    ),
)
def _emb_lookup(table_hbm, idx_hbm, out_hbm, idx_all, rows_v, *sems):
    _body(table_hbm, idx_hbm, out_hbm, idx_all, rows_v, *sems)


def kernel(inputs, embedding_weights):
    idx = inputs.astype(jnp.int32).reshape(B)
    out = _emb_lookup(embedding_weights, idx)
    return out.reshape(BATCH, SEQ, D)


# speedup vs baseline: 9.2482x; 1.0045x over previous
import functools

import jax
import jax.numpy as jnp
from jax import lax
from jax.experimental import pallas as pl
from jax.experimental.pallas import tpu as pltpu
from jax.experimental.pallas import tpu_sc as plsc

VOCAB = 100000
D = 128
BATCH = 4096
SEQ = 200
B = BATCH * SEQ
CHUNK = 160
G = B // CHUNK

_info = plsc.get_sparse_core_info()
NC, NS, L = _info.num_cores, _info.num_subcores, _info.num_lanes
NW = NC * NS
GPW = G // NW
RPW = GPW * CHUNK

NBUF = 4
LOOK = NBUF - 1


def _body(table_hbm, idx_hbm, out_hbm, idx_all, rows_v, *sems):
    gsems, osems = sems[:NBUF], sems[NBUF:]
    wid = lax.axis_index("s") * NC + lax.axis_index("c")
    rbase = wid * RPW

    pltpu.sync_copy(idx_hbm.at[pl.ds(rbase, RPW)], idx_all)

    def fire_gather(g, b):
        isl = idx_all.at[pl.ds(g * CHUNK, CHUNK)]
        pltpu.async_copy(table_hbm.at[isl], rows_v.at[b], gsems[b])

    def wait_gather(b):
        pltpu.make_async_copy(
            table_hbm.at[pl.ds(0, CHUNK)], rows_v.at[b], gsems[b]).wait()

    def fire_out(g, b):
        osl = out_hbm.at[pl.ds(rbase + g * CHUNK, CHUNK)]
        pltpu.async_copy(rows_v.at[b], osl, osems[b])

    def wait_out(b):
        pltpu.make_async_copy(
            rows_v.at[b], out_hbm.at[pl.ds(0, CHUNK)], osems[b]).wait()

    def fix(g, b):
        def grp_body(i, _):
            iv = idx_all[pl.ds(g * CHUNK + i * L, L)]
            nzero = plsc.all_reduce_population_count(iv == 0)[0]

            @pl.when(nzero > 0)
            def _fix():
                z = jnp.zeros((L,), jnp.float32)
                for r in range(L):

                    @pl.when(iv[r] == 0)
                    def _zero_row(row=i * L + r):
                        for c in range(D // L):
                            rows_v[b, row, pl.ds(c * L, L)] = z

            return 0

        lax.fori_loop(0, CHUNK // L, grp_body, 0)

    for b in range(LOOK):
        fire_gather(jnp.int32(b), b)

    def lap(k, _):
        for b in range(NBUF):
            g = k * NBUF + b
            wait_gather(b)
            fix(g, b)
            fire_out(g, b)
            gn = g + LOOK
            b2 = (b + LOOK) % NBUF

            @pl.when(gn < GPW)
            def _issue(gn=gn, b2=b2):
                @pl.when(gn >= NBUF)
                def _drain():
                    wait_out(b2)

                fire_gather(gn, b2)

        return 0

    lax.fori_loop(0, GPW // NBUF, lap, 0)

    for b in range(NBUF):
        wait_out(b)


@functools.partial(
    pl.kernel,
    out_type=jax.ShapeDtypeStruct((B, D), jnp.float32),
    mesh=plsc.VectorSubcoreMesh(core_axis_name="c", subcore_axis_name="s"),
    compiler_params=pltpu.CompilerParams(needs_layout_passes=False),
    scratch_types=(
        [pltpu.VMEM((RPW,), jnp.int32),
         pltpu.VMEM((NBUF, CHUNK, D), jnp.float32)]
        + [pltpu.SemaphoreType.DMA] * (2 * NBUF)
    ),
)
def _emb_lookup(table_hbm, idx_hbm, out_hbm, idx_all, rows_v, *sems):
    _body(table_hbm, idx_hbm, out_hbm, idx_all, rows_v, *sems)


def kernel(inputs, embedding_weights):
    idx = inputs.astype(jnp.int32).reshape(B)
    out = _emb_lookup(embedding_weights, idx)
    return out.reshape(BATCH, SEQ, D)
